# baseline (speedup 1.0000x reference)
import numpy as np
import jax
import jax.numpy as jnp
from jax import lax
from jax.experimental import pallas as pl
from jax.experimental.pallas import tpu as pltpu

_B, _L, _CIN, _D, _S, _TR = 16, 1024, 64, 128, 10, 512
_NU = _L - _TR


def _pe_np(max_len, d):
    position = np.arange(max_len)[:, None].astype(np.float32)
    div_term = np.exp(np.arange(0, d, 2).astype(np.float32) * -(np.log(10000.0) / d))
    pe = np.zeros((max_len, d), np.float32)
    pe[:, 0::2] = np.sin(position * div_term)
    pe[:, 1::2] = np.cos(position * div_term)
    return pe


_PE = _pe_np(5000, _D)[:_L]
_DENOM = np.minimum(np.arange(_L) + 1, _S).astype(np.float32)[:, None]

_LKEYS = ('Wq', 'bq', 'Wk', 'bk', 'Wv', 'bv', 'Wo', 'bo',
          'g1', 'b1', 'W1', 'c1', 'W2', 'c2', 'g2', 'b2')


def _dot(a, b):
    return jnp.dot(a, b, preferred_element_type=jnp.float32)


def _ln(x, g, b):
    m = jnp.mean(x, axis=-1, keepdims=True)
    c = x - m
    v = jnp.mean(c * c, axis=-1, keepdims=True)
    return c * lax.rsqrt(v + 1e-5) * g + b


def _softmax_ax(z, axis):
    m = jnp.max(z, axis=axis, keepdims=True)
    e = jnp.exp(z - m)
    return e / jnp.sum(e, axis=axis, keepdims=True)


def _attn_common(x, P):
    qkv = _dot(x, P[0]) + P[1]
    q = qkv[:, :_D]
    k = qkv[:, _D:2 * _D]
    v = qkv[:, 2 * _D:]
    lg = lax.dot_general(q, k, (((1,), (1,)), ((), ()))) * (1.0 / np.sqrt(_D))
    a = _softmax_ax(lg, 1)
    o = _dot(a, v)
    x = _ln(x + _dot(o, P[2]) + P[3], P[4], P[5])
    h = _dot(jnp.maximum(_dot(x, P[6]) + P[7], 0.0), P[8]) + P[9]
    return _ln(x + h, P[10], P[11]), a


def _enc_layer(x, P):
    return _attn_common(x, P)[0]


def _dec_layer(x, P):
    return _attn_common(x, P)


def _body(*refs):
    (x_ref, wc0, wc1, wc2, pe_ref, den_ref), rest = refs[:6], refs[6:]
    layers = []
    for i in range(4):
        layers.append([r[...] for r in rest[12 * i:12 * (i + 1)]])
    (egN, ebN, dgN, dbN, mtok, pw1, pb1, pw2, pb2) = [r[...] for r in rest[48:57]]
    a1_ref, a2_ref, rec_ref = rest[57:]

    x = x_ref[0]
    pe = pe_ref[...]
    xm1 = jnp.concatenate([x[-1:], x[:-1]], axis=0)
    xp1 = jnp.concatenate([x[1:], x[:1]], axis=0)
    ve = _dot(xm1, wc0[...]) + _dot(x, wc1[...]) + _dot(xp1, wc2[...])
    ex = ve + pe

    r2 = jnp.sum(ex * ex, axis=1, keepdims=True)
    w = ex
    s2 = r2
    for s in range(1, _S):
        zD = jnp.zeros((s, _D), jnp.float32)
        z1 = jnp.zeros((s, 1), jnp.float32)
        w = w + jnp.concatenate([zD, ex[:-s]], axis=0)
        s2 = s2 + jnp.concatenate([z1, r2[:-s]], axis=0)
    den = den_ref[...]
    sum_w = jnp.sum(w, axis=1, keepdims=True)
    sum_w2 = jnp.sum(w * w, axis=1, keepdims=True)
    num = s2 / den - sum_w2 / (den * den)
    score = num / (sum_w / den)

    kb = lax.bitcast_convert_type(score, jnp.uint32)
    keys = jnp.where((kb >> 31) != 0, ~kb, kb | jnp.uint32(0x80000000))
    keys8 = keys.reshape(8, 128)

    def sel(i, t):
        cand = t | lax.shift_left(jnp.uint32(1), (31 - i).astype(jnp.uint32))
        cnt = jnp.sum((keys8 >= cand).astype(jnp.int32))
        return jnp.where(cnt >= _TR, cand, t)

    thr = lax.fori_loop(0, 32, sel, jnp.uint32(0))
    maskd = keys >= thr

    unm = jnp.where(maskd, jnp.float32(0.0), jnp.float32(1.0))
    c = unm
    for st in (1, 2, 4, 8, 16, 32, 64, 128, 256, 512):
        c = c + jnp.concatenate([jnp.zeros((st, 1), jnp.float32), c[:-st]], 0)
    pos = c - unm
    jrow = lax.broadcasted_iota(jnp.int32, (1, _NU), 1).astype(jnp.float32)
    pt = jnp.where((pos == jrow) & (unm > 0.5), jnp.float32(1.0),
                   jnp.float32(0.0))

    xc = lax.dot_general(pt, ex, (((0,), (0,)), ((), ())))
    h = _enc_layer(xc, layers[0])
    h = _enc_layer(h, layers[1])
    h = _ln(h, egN, ebN)
    full = _dot(pt, h)

    tokens = jnp.where(maskd, mtok + pe, full)

    h, a1 = _dec_layer(tokens, layers[2])
    a1_ref[0] = a1
    h, a2 = _dec_layer(h, layers[3])
    a2_ref[0] = a2
    h = _ln(h, dgN, dbN)

    h = _dot(h, pw1) + pb1
    h = jax.nn.gelu(h)
    rec_ref[0] = jax.nn.sigmoid(_dot(h, pw2) + pb2)


def _full_spec(a):
    nd = a.ndim
    return pl.BlockSpec(a.shape, lambda b, _nd=nd: (0,) * _nd)


def kernel(x, params):
    p = params
    wc = p['Wc']
    consts = [wc[:, :, 0].T, wc[:, :, 1].T, wc[:, :, 2].T, _PE, _DENOM]
    for lyr in (*p['enc_layers'], *p['dec_layers']):
        consts.append(jnp.concatenate([lyr['Wq'], lyr['Wk'], lyr['Wv']], axis=1))
        consts.append(jnp.concatenate([lyr['bq'], lyr['bk'], lyr['bv']])[None, :])
        for kname in ('Wo', 'bo', 'g1', 'b1', 'W1', 'c1', 'W2', 'c2', 'g2', 'b2'):
            a = lyr[kname]
            consts.append(a[None, :] if a.ndim == 1 else a)
    consts += [p['enc_gN'][None], p['enc_bN'][None],
               p['dec_gN'][None], p['dec_bN'][None],
               p['mask_token'].reshape(1, _D),
               p['pw1'], p['pb1'][None], p['pw2'], p['pb2'][None]]

    in_specs = [pl.BlockSpec((1, _L, _CIN), lambda b: (b, 0, 0))]
    in_specs += [_full_spec(a) for a in consts]
    out_specs = [pl.BlockSpec((1, _L, _L), lambda b: (b, 0, 0)),
                 pl.BlockSpec((1, _L, _L), lambda b: (b, 0, 0)),
                 pl.BlockSpec((1, _L, _D), lambda b: (b, 0, 0))]
    out_shape = [jax.ShapeDtypeStruct((_B, _L, _L), jnp.float32),
                 jax.ShapeDtypeStruct((_B, _L, _L), jnp.float32),
                 jax.ShapeDtypeStruct((_B, _L, _D), jnp.float32)]

    a1, a2, rec = pl.pallas_call(
        _body,
        grid=(_B,),
        in_specs=in_specs,
        out_specs=out_specs,
        out_shape=out_shape,
        compiler_params=pltpu.CompilerParams(
            dimension_semantics=("arbitrary",)),
    )(x, *consts)
    return a1, a2, rec


# speedup vs baseline: 1.1716x; 1.1716x over previous
import numpy as np
import jax
import jax.numpy as jnp
from jax import lax
from jax.experimental import pallas as pl
from jax.experimental.pallas import tpu as pltpu

_B, _L, _CIN, _D, _S, _TR = 16, 1024, 64, 128, 10, 512
_NU = _L - _TR


def _pe_np(max_len, d):
    position = np.arange(max_len)[:, None].astype(np.float32)
    div_term = np.exp(np.arange(0, d, 2).astype(np.float32) * -(np.log(10000.0) / d))
    pe = np.zeros((max_len, d), np.float32)
    pe[:, 0::2] = np.sin(position * div_term)
    pe[:, 1::2] = np.cos(position * div_term)
    return pe


_PE = _pe_np(5000, _D)[:_L]
_DENOM = np.minimum(np.arange(_L) + 1, _S).astype(np.float32)[:, None]

_LKEYS = ('Wq', 'bq', 'Wk', 'bk', 'Wv', 'bv', 'Wo', 'bo',
          'g1', 'b1', 'W1', 'c1', 'W2', 'c2', 'g2', 'b2')


def _dot(a, b):
    return jnp.dot(a, b, preferred_element_type=jnp.float32)


def _ln(x, g, b):
    m = jnp.mean(x, axis=-1, keepdims=True)
    c = x - m
    v = jnp.mean(c * c, axis=-1, keepdims=True)
    return c * lax.rsqrt(v + 1e-5) * g + b


def _softmax_ax(z, axis):
    m = jnp.max(z, axis=axis, keepdims=True)
    e = jnp.exp(z - m)
    return e / jnp.sum(e, axis=axis, keepdims=True)


def _attn_common(x, P):
    qkv = _dot(x, P[0]) + P[1]
    q = qkv[:, :_D]
    k = qkv[:, _D:2 * _D]
    v = qkv[:, 2 * _D:]
    lg = lax.dot_general(q, k, (((1,), (1,)), ((), ()))) * (1.0 / np.sqrt(_D))
    a = _softmax_ax(lg, 1)
    o = _dot(a, v)
    x = _ln(x + _dot(o, P[2]) + P[3], P[4], P[5])
    h = _dot(jnp.maximum(_dot(x, P[6]) + P[7], 0.0), P[8]) + P[9]
    return _ln(x + h, P[10], P[11]), a


def _enc_layer(x, P):
    return _attn_common(x, P)[0]


def _dec_layer(x, P):
    return _attn_common(x, P)


def _body(*refs):
    (x_ref, wc0, wc1, wc2, pe_ref, den_ref), rest = refs[:6], refs[6:]
    layers = []
    for i in range(4):
        layers.append([r[...] for r in rest[12 * i:12 * (i + 1)]])
    (egN, ebN, dgN, dbN, mtok, pw1, pb1, pw2, pb2) = [r[...] for r in rest[48:57]]
    a1_ref, a2_ref, rec_ref = rest[57:]

    x = x_ref[0]
    pe = pe_ref[...]
    xm1 = jnp.concatenate([x[-1:], x[:-1]], axis=0)
    xp1 = jnp.concatenate([x[1:], x[:1]], axis=0)
    ve = _dot(xm1, wc0[...]) + _dot(x, wc1[...]) + _dot(xp1, wc2[...])
    ex = ve + pe

    r2 = jnp.sum(ex * ex, axis=1, keepdims=True)
    w = ex
    s2 = r2
    for s in range(1, _S):
        zD = jnp.zeros((s, _D), jnp.float32)
        z1 = jnp.zeros((s, 1), jnp.float32)
        w = w + jnp.concatenate([zD, ex[:-s]], axis=0)
        s2 = s2 + jnp.concatenate([z1, r2[:-s]], axis=0)
    den = den_ref[...]
    sum_w = jnp.sum(w, axis=1, keepdims=True)
    sum_w2 = jnp.sum(w * w, axis=1, keepdims=True)
    num = s2 / den - sum_w2 / (den * den)
    score = num / (sum_w / den)

    kb = lax.bitcast_convert_type(score, jnp.uint32)
    keys = jnp.where((kb >> 31) != 0, ~kb, kb | jnp.uint32(0x80000000))
    keys8 = keys

    def sel(i, t):
        cand = t | lax.shift_left(jnp.uint32(1), (31 - i).astype(jnp.uint32))
        cnt = jnp.sum((keys8 >= cand).astype(jnp.int32))
        return jnp.where(cnt >= _TR, cand, t)

    thr = lax.fori_loop(0, 32, sel, jnp.uint32(0))
    maskd = keys >= thr

    unm = jnp.where(maskd, jnp.float32(0.0), jnp.float32(1.0))
    c = unm
    for st in (1, 2, 4, 8, 16, 32, 64, 128, 256, 512):
        c = c + jnp.concatenate([jnp.zeros((st, 1), jnp.float32), c[:-st]], 0)
    pos = c - unm
    jrow = lax.broadcasted_iota(jnp.int32, (1, _NU), 1).astype(jnp.float32)
    pt = jnp.where((pos == jrow) & (unm > 0.5), jnp.float32(1.0),
                   jnp.float32(0.0))

    xc = lax.dot_general(pt, ex, (((0,), (0,)), ((), ())))
    h = _enc_layer(xc, layers[0])
    h = _enc_layer(h, layers[1])
    h = _ln(h, egN, ebN)
    full = _dot(pt, h)

    tokens = jnp.where(maskd, mtok + pe, full)

    h, a1 = _dec_layer(tokens, layers[2])
    a1_ref[0] = a1
    h, a2 = _dec_layer(h, layers[3])
    a2_ref[0] = a2
    h = _ln(h, dgN, dbN)

    h = _dot(h, pw1) + pb1
    h = jax.nn.gelu(h)
    rec_ref[0] = jax.nn.sigmoid(_dot(h, pw2) + pb2)


def _full_spec(a):
    nd = a.ndim
    return pl.BlockSpec(a.shape, lambda b, _nd=nd: (0,) * _nd)


def kernel(x, params):
    p = params
    wc = p['Wc']
    consts = [wc[:, :, 0].T, wc[:, :, 1].T, wc[:, :, 2].T, _PE, _DENOM]
    for lyr in (*p['enc_layers'], *p['dec_layers']):
        consts.append(jnp.concatenate([lyr['Wq'], lyr['Wk'], lyr['Wv']], axis=1))
        consts.append(jnp.concatenate([lyr['bq'], lyr['bk'], lyr['bv']])[None, :])
        for kname in ('Wo', 'bo', 'g1', 'b1', 'W1', 'c1', 'W2', 'c2', 'g2', 'b2'):
            a = lyr[kname]
            consts.append(a[None, :] if a.ndim == 1 else a)
    consts += [p['enc_gN'][None], p['enc_bN'][None],
               p['dec_gN'][None], p['dec_bN'][None],
               p['mask_token'].reshape(1, _D),
               p['pw1'], p['pb1'][None], p['pw2'], p['pb2'][None]]

    in_specs = [pl.BlockSpec((1, _L, _CIN), lambda b: (b, 0, 0))]
    in_specs += [_full_spec(a) for a in consts]
    out_specs = [pl.BlockSpec((1, _L, _L), lambda b: (b, 0, 0)),
                 pl.BlockSpec((1, _L, _L), lambda b: (b, 0, 0)),
                 pl.BlockSpec((1, _L, _D), lambda b: (b, 0, 0))]
    out_shape = [jax.ShapeDtypeStruct((_B, _L, _L), jnp.float32),
                 jax.ShapeDtypeStruct((_B, _L, _L), jnp.float32),
                 jax.ShapeDtypeStruct((_B, _L, _D), jnp.float32)]

    a1, a2, rec = pl.pallas_call(
        _body,
        grid=(_B,),
        in_specs=in_specs,
        out_specs=out_specs,
        out_shape=out_shape,
        compiler_params=pltpu.CompilerParams(
            dimension_semantics=("arbitrary",)),
    )(x, *consts)
    return a1, a2, rec


# speedup vs baseline: 1.3542x; 1.1559x over previous
import numpy as np
import jax
import jax.numpy as jnp
from jax import lax
from jax.experimental import pallas as pl
from jax.experimental.pallas import tpu as pltpu

_B, _L, _CIN, _D, _S, _TR = 16, 1024, 64, 128, 10, 512
_NU = _L - _TR


def _pe_np(max_len, d):
    position = np.arange(max_len)[:, None].astype(np.float32)
    div_term = np.exp(np.arange(0, d, 2).astype(np.float32) * -(np.log(10000.0) / d))
    pe = np.zeros((max_len, d), np.float32)
    pe[:, 0::2] = np.sin(position * div_term)
    pe[:, 1::2] = np.cos(position * div_term)
    return pe


_PE = _pe_np(5000, _D)[:_L]
_DENOM = np.minimum(np.arange(_L) + 1, _S).astype(np.float32)[:, None]

_LKEYS = ('Wq', 'bq', 'Wk', 'bk', 'Wv', 'bv', 'Wo', 'bo',
          'g1', 'b1', 'W1', 'c1', 'W2', 'c2', 'g2', 'b2')


def _dot(a, b):
    return jnp.dot(a, b, preferred_element_type=jnp.float32)


def _ln(x, g, b):
    m = jnp.mean(x, axis=-1, keepdims=True)
    c = x - m
    v = jnp.mean(c * c, axis=-1, keepdims=True)
    return c * lax.rsqrt(v + 1e-5) * g + b


def _softmax_ax(z, axis):
    m = jnp.max(z, axis=axis, keepdims=True)
    e = jnp.exp(z - m)
    return e / jnp.sum(e, axis=axis, keepdims=True)


def _attn_common(x, P):
    qkv = _dot(x, P[0]) + P[1]
    q = qkv[:, :_D]
    k = qkv[:, _D:2 * _D]
    v = qkv[:, 2 * _D:]
    lg = lax.dot_general(q, k, (((1,), (1,)), ((), ()))) * (1.0 / np.sqrt(_D))
    a = _softmax_ax(lg, 1)
    o = _dot(a, v)
    x = _ln(x + _dot(o, P[2]) + P[3], P[4], P[5])
    h = _dot(jnp.maximum(_dot(x, P[6]) + P[7], 0.0), P[8]) + P[9]
    return _ln(x + h, P[10], P[11]), a


def _enc_layer(x, P):
    return _attn_common(x, P)[0]


def _dec_layer(x, P):
    return _attn_common(x, P)


def _body(*refs):
    (x_ref, wc0, wc1, wc2, pe_ref, den_ref), rest = refs[:6], refs[6:]
    layers = []
    for i in range(4):
        layers.append([r[...] for r in rest[12 * i:12 * (i + 1)]])
    (egN, ebN, dgN, dbN, mtok, pw1, pb1, pw2, pb2) = [r[...] for r in rest[48:57]]
    a1_ref, a2_ref, rec_ref = rest[57:]

    x = x_ref[0]
    pe = pe_ref[...]
    xm1 = jnp.concatenate([x[-1:], x[:-1]], axis=0)
    xp1 = jnp.concatenate([x[1:], x[:1]], axis=0)
    ve = _dot(xm1, wc0[...]) + _dot(x, wc1[...]) + _dot(xp1, wc2[...])
    ex = ve + pe

    r2 = jnp.sum(ex * ex, axis=1, keepdims=True)
    w = ex
    s2 = r2
    for s in range(1, _S):
        zD = jnp.zeros((s, _D), jnp.float32)
        z1 = jnp.zeros((s, 1), jnp.float32)
        w = w + jnp.concatenate([zD, ex[:-s]], axis=0)
        s2 = s2 + jnp.concatenate([z1, r2[:-s]], axis=0)
    den = den_ref[...]
    sum_w = jnp.sum(w, axis=1, keepdims=True)
    sum_w2 = jnp.sum(w * w, axis=1, keepdims=True)
    num = s2 / den - sum_w2 / (den * den)
    score = num / (sum_w / den)

    kb = lax.bitcast_convert_type(score, jnp.uint32)
    keys = jnp.where((kb >> 31) != 0, ~kb, kb | jnp.uint32(0x80000000))

    iot = lax.broadcasted_iota(jnp.uint32, (1, 256), 1)
    iot_f = iot.astype(jnp.float32)
    prefix = jnp.zeros((1, 1), jnp.uint32)
    kcur = jnp.full((1, 1), _TR, jnp.float32)
    for shift in (24, 16, 8, 0):
        byte = lax.shift_right_logical(keys, jnp.uint32(shift)) & jnp.uint32(0xFF)
        hit = byte == iot
        if shift < 24:
            hi = lax.shift_right_logical(keys, jnp.uint32(shift + 8))
            hit = hit & (hi == prefix)
        oh = jnp.where(hit, jnp.float32(1.0), jnp.float32(0.0))
        cnt = jnp.sum(oh, axis=0, keepdims=True)
        c = cnt
        for st in (1, 2, 4, 8, 16, 32, 64, 128):
            c = c + jnp.concatenate(
                [jnp.zeros((1, st), jnp.float32), c[:, :-st]], axis=1)
        total = jnp.sum(cnt, axis=1, keepdims=True)
        csum = total - c + cnt
        bst = jnp.max(jnp.where(csum >= kcur, iot_f, jnp.float32(-1.0)),
                      axis=1, keepdims=True)
        above = jnp.sum(jnp.where(iot_f == bst, csum - cnt, jnp.float32(0.0)),
                        axis=1, keepdims=True)
        kcur = kcur - above
        prefix = lax.shift_left(prefix, jnp.uint32(8)) | bst.astype(jnp.uint32)
    maskd = keys >= prefix

    unm = jnp.where(maskd, jnp.float32(0.0), jnp.float32(1.0))
    c = unm
    for st in (1, 2, 4, 8, 16, 32, 64, 128, 256, 512):
        c = c + jnp.concatenate([jnp.zeros((st, 1), jnp.float32), c[:-st]], 0)
    pos = c - unm
    jrow = lax.broadcasted_iota(jnp.int32, (1, _NU), 1).astype(jnp.float32)
    pt = jnp.where((pos == jrow) & (unm > 0.5), jnp.float32(1.0),
                   jnp.float32(0.0))

    xc = lax.dot_general(pt, ex, (((0,), (0,)), ((), ())))
    h = _enc_layer(xc, layers[0])
    h = _enc_layer(h, layers[1])
    h = _ln(h, egN, ebN)
    full = _dot(pt, h)

    tokens = jnp.where(maskd, mtok + pe, full)

    h, a1 = _dec_layer(tokens, layers[2])
    a1_ref[0] = a1
    h, a2 = _dec_layer(h, layers[3])
    a2_ref[0] = a2
    h = _ln(h, dgN, dbN)

    h = _dot(h, pw1) + pb1
    h = jax.nn.gelu(h)
    rec_ref[0] = jax.nn.sigmoid(_dot(h, pw2) + pb2)


def _full_spec(a):
    nd = a.ndim
    return pl.BlockSpec(a.shape, lambda b, _nd=nd: (0,) * _nd)


def kernel(x, params):
    p = params
    wc = p['Wc']
    consts = [wc[:, :, 0].T, wc[:, :, 1].T, wc[:, :, 2].T, _PE, _DENOM]
    for lyr in (*p['enc_layers'], *p['dec_layers']):
        consts.append(jnp.concatenate([lyr['Wq'], lyr['Wk'], lyr['Wv']], axis=1))
        consts.append(jnp.concatenate([lyr['bq'], lyr['bk'], lyr['bv']])[None, :])
        for kname in ('Wo', 'bo', 'g1', 'b1', 'W1', 'c1', 'W2', 'c2', 'g2', 'b2'):
            a = lyr[kname]
            consts.append(a[None, :] if a.ndim == 1 else a)
    consts += [p['enc_gN'][None], p['enc_bN'][None],
               p['dec_gN'][None], p['dec_bN'][None],
               p['mask_token'].reshape(1, _D),
               p['pw1'], p['pb1'][None], p['pw2'], p['pb2'][None]]

    in_specs = [pl.BlockSpec((1, _L, _CIN), lambda b: (b, 0, 0))]
    in_specs += [_full_spec(a) for a in consts]
    out_specs = [pl.BlockSpec((1, _L, _L), lambda b: (b, 0, 0)),
                 pl.BlockSpec((1, _L, _L), lambda b: (b, 0, 0)),
                 pl.BlockSpec((1, _L, _D), lambda b: (b, 0, 0))]
    out_shape = [jax.ShapeDtypeStruct((_B, _L, _L), jnp.float32),
                 jax.ShapeDtypeStruct((_B, _L, _L), jnp.float32),
                 jax.ShapeDtypeStruct((_B, _L, _D), jnp.float32)]

    a1, a2, rec = pl.pallas_call(
        _body,
        grid=(_B,),
        in_specs=in_specs,
        out_specs=out_specs,
        out_shape=out_shape,
        compiler_params=pltpu.CompilerParams(
            dimension_semantics=("arbitrary",)),
    )(x, *consts)
    return a1, a2, rec
